# Initial kernel scaffold; baseline (speedup 1.0000x reference)
import jax
import jax.numpy as jnp
from jax.experimental import pallas as pl


def kernel(x0, src0, dst0, src1, dst1, src2, dst2, src3, dst3, W01, b01, W02, b02, W03, b03, W11, b11, W12, b12, W21, b21, W22, b22, W3, b3):
    raise NotImplementedError("write your pallas kernel here")


# speedup vs baseline: 3.3258x; 3.3258x over previous
import functools

import jax
import jax.numpy as jnp
from jax import lax
from jax.experimental import pallas as pl
from jax.experimental.pallas import tpu as pltpu
from jax.experimental.pallas import tpu_sc as plsc

_F32 = jnp.float32
_NC = 2
_NS = 16
_EB = 128

_N0, _N1, _N2, _N3, _N4 = 100000, 40000, 16000, 6400, 2560


def _sc_mesh():
    return plsc.VectorSubcoreMesh(
        core_axis_name="c", subcore_axis_name="s",
        num_cores=_NC, num_subcores=_NS)


def _row_part(n):
    rpt = -(-(-(-n // _NS)) // 8) * 8
    last = n - (_NS - 1) * rpt
    assert last > 0
    return rpt, last


def _tiled_copy(s, n, copy_fn):
    rpt, last = _row_part(n)
    if last == rpt:
        copy_fn(s * rpt, rpt)
    else:
        @pl.when(s < _NS - 1)
        def _():
            copy_fn(s * rpt, rpt)

        @pl.when(s == _NS - 1)
        def _():
            copy_fn((_NS - 1) * rpt, last)


def _sc_degrees(idx_arrays, n_nodes):
    n_arr = len(idx_arrays)
    half = n_arr // 2
    splits = (tuple(range(half)), tuple(range(half, n_arr)))
    max_n = max(n_nodes)
    max_rpt = max(_row_part(n)[0] for n in n_nodes)
    zeros = jnp.zeros((max_rpt, 16), _F32)
    ones_blk = (lax.broadcasted_iota(jnp.int32, (_EB, 16), 1) == 0).astype(_F32)

    def body(*refs):
        idx_hbm = refs[0:n_arr]
        zeros_hbm = refs[n_arr]
        ones_hbm = refs[n_arr + 1]
        outs = refs[n_arr + 2:n_arr + 2 + n_arr]
        acc, idx_v, ones_v = refs[n_arr + 2 + n_arr:]
        c = lax.axis_index("c")
        s = lax.axis_index("s")
        pltpu.sync_copy(ones_hbm, ones_v)
        for ci, passes in enumerate(splits):
            @pl.when(c == ci)
            def _(passes=passes):
                for p in passes:
                    n = n_nodes[p]
                    nblk = idx_arrays[p].shape[0] // _EB
                    per = -(-nblk // _NS)
                    _tiled_copy(s, n, lambda r0, nr: pltpu.sync_copy(
                        zeros_hbm.at[pl.ds(0, nr)], acc.at[pl.ds(r0, nr)]))
                    plsc.subcore_barrier()

                    def blk(i, carry, p=p, nblk=nblk):
                        b = i * _NS + s
                        @pl.when(b < nblk)
                        def _():
                            pltpu.sync_copy(
                                idx_hbm[p].at[pl.ds(b * _EB, _EB)], idx_v)
                            pltpu.sync_copy(ones_v, acc.at[idx_v], add=True)
                        return carry

                    lax.fori_loop(0, per, blk, 0)
                    plsc.subcore_barrier()
                    _tiled_copy(s, n, lambda r0, nr, p=p: pltpu.sync_copy(
                        acc.at[pl.ds(r0, nr)], outs[p].at[pl.ds(r0, nr)]))
                    plsc.subcore_barrier()

    kfn = pl.kernel(
        body,
        out_type=[jax.ShapeDtypeStruct((n, 16), _F32) for n in n_nodes],
        mesh=_sc_mesh(),
        compiler_params=pltpu.CompilerParams(use_tc_tiling_on_sc=False),
        scratch_types=[
            pltpu.VMEM_SHARED((max_n, 16), _F32),
            pltpu.VMEM((_EB,), jnp.int32),
            pltpu.VMEM((_EB, 16), _F32),
        ],
    )
    return kfn(*idx_arrays, zeros, ones_blk)


def _sc_gather_scatter(src, dst, tables, n_dst, width):
    k = len(tables)
    half = k // 2
    splits = (tuple(range(half)), tuple(range(half, k)))
    nblk = src.shape[0] // _EB
    per = -(-nblk // _NS)
    zeros = jnp.zeros((_row_part(n_dst)[0], width), _F32)

    def body(*refs):
        src_hbm, dst_hbm = refs[0], refs[1]
        tabs = refs[2:2 + k]
        zeros_hbm = refs[2 + k]
        outs = refs[3 + k:3 + 2 * k]
        acc, idx_s, idx_d, gbuf, sem = refs[3 + 2 * k:]
        c = lax.axis_index("c")
        s = lax.axis_index("s")
        for ci, chunks in enumerate(splits):
            @pl.when(c == ci)
            def _(chunks=chunks):
                for ch in chunks:
                    _tiled_copy(s, n_dst, lambda r0, nr: pltpu.sync_copy(
                        zeros_hbm.at[pl.ds(0, nr)], acc.at[pl.ds(r0, nr)]))
                    plsc.subcore_barrier()

                    def blk(i, carry, ch=ch):
                        b = i * _NS + s
                        @pl.when(b < nblk)
                        def _():
                            pltpu.sync_copy(
                                src_hbm.at[pl.ds(b * _EB, _EB)], idx_s)
                            pltpu.sync_copy(
                                dst_hbm.at[pl.ds(b * _EB, _EB)], idx_d)
                            pltpu.async_copy(
                                tabs[ch].at[idx_s], gbuf, sem).wait()
                            pltpu.sync_copy(gbuf, acc.at[idx_d], add=True)
                        return carry

                    lax.fori_loop(0, per, blk, 0)
                    plsc.subcore_barrier()
                    _tiled_copy(s, n_dst, lambda r0, nr, ch=ch: pltpu.sync_copy(
                        acc.at[pl.ds(r0, nr)], outs[ch].at[pl.ds(r0, nr)]))
                    plsc.subcore_barrier()

    kfn = pl.kernel(
        body,
        out_type=[jax.ShapeDtypeStruct((n_dst, width), _F32)
                  for _ in range(k)],
        mesh=_sc_mesh(),
        compiler_params=pltpu.CompilerParams(use_tc_tiling_on_sc=False),
        scratch_types=[
            pltpu.VMEM_SHARED((n_dst, width), _F32),
            pltpu.VMEM((_EB,), jnp.int32),
            pltpu.VMEM((_EB,), jnp.int32),
            pltpu.VMEM((_EB, width), _F32),
            pltpu.SemaphoreType.DMA,
        ],
    )
    return kfn(src, dst, *tables, zeros)


def _rs(col):
    return lax.rsqrt(jnp.maximum(col, 1.0))


def _tc_prescale0(x0, deg_src0):
    n = x0.shape[0]
    br = 1000

    def body(x_ref, dg_ref, o0, o1, o2, o3):
        sc = _rs(dg_ref[:, 0])[:, None]
        outs = (o0, o1, o2, o3)
        for ci in range(4):
            outs[ci][...] = x_ref[:, ci * 32:(ci + 1) * 32] * sc

    return pl.pallas_call(
        body,
        grid=(n // br,),
        in_specs=[pl.BlockSpec((br, 128), lambda i: (i, 0)),
                  pl.BlockSpec((br, 16), lambda i: (i, 0))],
        out_specs=[pl.BlockSpec((br, 32), lambda i: (i, 0))] * 4,
        out_shape=[jax.ShapeDtypeStruct((n, 32), _F32)] * 4,
    )(x0, deg_src0)


def _tc_layer1(aggs, deg_dst0, deg_src1, Wcat, bcat):
    n = aggs[0].shape[0]
    br = 1000

    def body(a0, a1, a2, a3, dd, dsr, W, bb, y0, y1, y2, y3, s12, s13):
        i = pl.program_id(0)
        X = jnp.concatenate([a0[...], a1[...], a2[...], a3[...]], axis=1)
        sd = _rs(dd[:, 0])[:, None]
        H = jnp.dot(X * sd, W[...], preferred_element_type=_F32) + bb[...]
        H = jnp.maximum(H, 0.0)
        x11 = H[:, :128]
        x12 = H[:, 128:256]
        x13 = H[:, 256:]
        sum1 = x11 + x12
        sc = _rs(dsr[:, 0])[:, None]
        y0[...] = sum1[:, :64] * sc
        y1[...] = sum1[:, 64:] * sc
        y2[...] = x12[:, :64] * sc
        y3[...] = x12[:, 64:] * sc

        @pl.when(i == 0)
        def _():
            s12[...] = jnp.zeros((1, 1), _F32)
            s13[...] = jnp.zeros((1, 1), _F32)
        s12[...] = s12[...] + jnp.sum(x12)
        s13[...] = s13[...] + jnp.sum(x13)

    scal = lambda i: (0, 0)
    return pl.pallas_call(
        body,
        grid=(n // br,),
        in_specs=[pl.BlockSpec((br, 32), lambda i: (i, 0))] * 4 + [
            pl.BlockSpec((br, 16), lambda i: (i, 0)),
            pl.BlockSpec((br, 16), lambda i: (i, 0)),
            pl.BlockSpec((128, 384), scal),
            pl.BlockSpec((1, 384), scal),
        ],
        out_specs=[pl.BlockSpec((br, 64), lambda i: (i, 0))] * 4 + [
            pl.BlockSpec((1, 1), scal), pl.BlockSpec((1, 1), scal)],
        out_shape=[jax.ShapeDtypeStruct((n, 64), _F32)] * 4 + [
            jax.ShapeDtypeStruct((1, 1), _F32)] * 2,
    )(*aggs, deg_dst0, deg_src1, Wcat, bcat)


def _tc_layer2(aggs, deg_dst1, deg_src2, W11, b11, W12, b12):
    n = aggs[0].shape[0]
    br = 1000

    def body(a0, a1, a2, a3, dd, dsr, Wa, ba, Wb, bb, z0, z1):
        sd = _rs(dd[:, 0])[:, None]
        AS = jnp.concatenate([a0[...], a1[...]], axis=1) * sd
        AX = jnp.concatenate([a2[...], a3[...]], axis=1) * sd
        x21 = jnp.maximum(
            jnp.dot(AS, Wa[...], preferred_element_type=_F32) + ba[...], 0.0)
        x22 = jnp.maximum(
            jnp.dot(AX, Wb[...], preferred_element_type=_F32) + bb[...], 0.0)
        sc = _rs(dsr[:, 0])[:, None]
        aux = jnp.concatenate([sc, jnp.zeros((br, 31), _F32)], axis=1)
        full = jnp.concatenate([x21 * sc, x22 * sc, aux], axis=1)
        z0[...] = full[:, :144]
        z1[...] = full[:, 144:]

    scal = lambda i: (0, 0)
    return pl.pallas_call(
        body,
        grid=(n // br,),
        in_specs=[pl.BlockSpec((br, 64), lambda i: (i, 0))] * 4 + [
            pl.BlockSpec((br, 16), lambda i: (i, 0)),
            pl.BlockSpec((br, 16), lambda i: (i, 0)),
            pl.BlockSpec((128, 128), scal), pl.BlockSpec((1, 128), scal),
            pl.BlockSpec((128, 128), scal), pl.BlockSpec((1, 128), scal),
        ],
        out_specs=[pl.BlockSpec((br, 144), lambda i: (i, 0))] * 2,
        out_shape=[jax.ShapeDtypeStruct((n, 144), _F32)] * 2,
    )(*aggs, deg_dst1, deg_src2, W11, b11, W12, b12)


def _tc_layer3(o0, o1, deg_dst2, deg_src3, W21, b21, W22, b22, s12):
    n = o0.shape[0]
    br = 800

    def body(r0, r1, dd, dsr, Wa, ba, Wb, bb, sm, w0, w1):
        cat = jnp.concatenate([r0[...], r1[...]], axis=1)
        sd = _rs(dd[:, 0])[:, None]
        aggA = cat[:, :128] * sd
        aggB = cat[:, 128:256] * sd
        n2 = cat[:, 256:257] * sd
        c1 = sm[...][0, 0] * (1.0 / (float(_N1) * 128.0))
        x31 = jnp.maximum(
            jnp.dot(aggA + aggB, Wa[...], preferred_element_type=_F32)
            + ba[...], 0.0)
        x32 = jnp.maximum(
            jnp.dot(aggA + c1 * n2, Wb[...], preferred_element_type=_F32)
            + bb[...], 0.0)
        S = x31 + x32
        sc3 = _rs(dsr[:, 0])[:, None]
        aux = jnp.concatenate([sc3, jnp.zeros((br, 31), _F32)], axis=1)
        full = jnp.concatenate([S * sc3, aux], axis=1)
        w0[...] = full[:, :80]
        w1[...] = full[:, 80:]

    scal = lambda i: (0, 0)
    return pl.pallas_call(
        body,
        grid=(n // br,),
        in_specs=[pl.BlockSpec((br, 144), lambda i: (i, 0))] * 2 + [
            pl.BlockSpec((br, 16), lambda i: (i, 0)),
            pl.BlockSpec((br, 16), lambda i: (i, 0)),
            pl.BlockSpec((128, 128), scal), pl.BlockSpec((1, 128), scal),
            pl.BlockSpec((128, 128), scal), pl.BlockSpec((1, 128), scal),
            pl.BlockSpec((1, 1), scal),
        ],
        out_specs=[pl.BlockSpec((br, 80), lambda i: (i, 0))] * 2,
        out_shape=[jax.ShapeDtypeStruct((n, 80), _F32)] * 2,
    )(o0, o1, deg_dst2, deg_src3, W21, b21, W22, b22, s12)


def _tc_final(d0, d1, deg_dst3, W3, b3, s13):
    n = d0.shape[0]

    def body(r0, r1, dd, W, bb, sm, o):
        cat = jnp.concatenate([r0[...], r1[...]], axis=1)
        sd = _rs(dd[:, 0])[:, None]
        F = cat[:, :128] * sd
        n3 = cat[:, 128:129] * sd
        c2 = sm[...][0, 0] * (1.0 / (float(_N1) * 128.0))
        o[...] = jnp.maximum(
            jnp.dot(F + c2 * n3, W[...], preferred_element_type=_F32)
            + bb[...], 0.0)

    scal = lambda: (0, 0)
    return pl.pallas_call(
        body,
        in_specs=[pl.BlockSpec((n, 80), scal)] * 2 + [
            pl.BlockSpec((n, 16), scal),
            pl.BlockSpec((128, 128), scal), pl.BlockSpec((1, 128), scal),
            pl.BlockSpec((1, 1), scal),
        ],
        out_specs=pl.BlockSpec((n, 128), scal),
        out_shape=jax.ShapeDtypeStruct((n, 128), _F32),
    )(d0, d1, deg_dst3, W3, b3, s13)


def kernel(x0, src0, dst0, src1, dst1, src2, dst2, src3, dst3,
           W01, b01, W02, b02, W03, b03, W11, b11, W12, b12,
           W21, b21, W22, b22, W3, b3):
    degs = _sc_degrees(
        [src0, src1, src2, src3, dst0, dst1, dst2, dst3],
        [_N0, _N1, _N2, _N3, _N1, _N2, _N3, _N4])
    dsrc0, dsrc1, dsrc2, dsrc3, ddst0, ddst1, ddst2, ddst3 = degs

    xt = _tc_prescale0(x0, dsrc0)
    agg0 = _sc_gather_scatter(src0, dst0, xt, _N1, 32)

    Wcat = jnp.concatenate([W01, W02, W03], axis=1)
    bcat = jnp.concatenate([b01, b02, b03]).reshape(1, 384)
    y0, y1, y2, y3, s12, s13 = _tc_layer1(agg0, ddst0, dsrc1, Wcat, bcat)

    agg1 = _sc_gather_scatter(src1, dst1, (y0, y1, y2, y3), _N2, 64)
    zt = _tc_layer2(agg1, ddst1, dsrc2,
                    W11, b11.reshape(1, 128), W12, b12.reshape(1, 128))

    agg2 = _sc_gather_scatter(src2, dst2, zt, _N3, 144)
    wt = _tc_layer3(agg2[0], agg2[1], ddst2, dsrc3,
                    W21, b21.reshape(1, 128), W22, b22.reshape(1, 128), s12)

    agg3 = _sc_gather_scatter(src3, dst3, wt, _N4, 80)
    return _tc_final(agg3[0], agg3[1], ddst3, W3, b3.reshape(1, 128), s13)


# speedup vs baseline: 5.5509x; 1.6691x over previous
import functools

import jax
import jax.numpy as jnp
from jax import lax
from jax.experimental import pallas as pl
from jax.experimental.pallas import tpu as pltpu
from jax.experimental.pallas import tpu_sc as plsc

_F32 = jnp.float32
_NC = 2
_NS = 16
_EB = 128

_N0, _N1, _N2, _N3, _N4 = 100000, 40000, 16000, 6400, 2560


def _sc_mesh():
    return plsc.VectorSubcoreMesh(
        core_axis_name="c", subcore_axis_name="s",
        num_cores=_NC, num_subcores=_NS)


def _row_part(n):
    rpt = -(-(-(-n // _NS)) // 8) * 8
    last = n - (_NS - 1) * rpt
    assert last > 0
    return rpt, last


def _tiled_copy(s, n, copy_fn):
    rpt, last = _row_part(n)
    if last == rpt:
        copy_fn(s * rpt, rpt)
    else:
        @pl.when(s < _NS - 1)
        def _():
            copy_fn(s * rpt, rpt)

        @pl.when(s == _NS - 1)
        def _():
            copy_fn((_NS - 1) * rpt, last)


def _sc_degrees(idx_arrays, n_nodes):
    n_arr = len(idx_arrays)
    half = n_arr // 2
    splits = (tuple(range(half)), tuple(range(half, n_arr)))
    max_n = max(n_nodes)
    max_rpt = max(_row_part(n)[0] for n in n_nodes)
    zeros = jnp.zeros((max_rpt, 16), _F32)
    ones_blk = (lax.broadcasted_iota(jnp.int32, (_EB, 16), 1) == 0).astype(_F32)

    K = 8

    def body(*refs):
        idx_hbm = refs[0:n_arr]
        zeros_hbm = refs[n_arr]
        ones_hbm = refs[n_arr + 1]
        outs = refs[n_arr + 2:n_arr + 2 + n_arr]
        acc = refs[2 * n_arr + 2]
        idx_v = refs[2 * n_arr + 3:2 * n_arr + 3 + K]
        ones_v, semi, sems = refs[2 * n_arr + 3 + K:]
        c = lax.axis_index("c")
        s = lax.axis_index("s")
        pltpu.sync_copy(ones_hbm, ones_v)
        for ci, passes in enumerate(splits):
            @pl.when(c == ci)
            def _(passes=passes):
                for p in passes:
                    n = n_nodes[p]
                    nblk = idx_arrays[p].shape[0] // _EB
                    per = -(-nblk // _NS)
                    nbat = -(-per // K)
                    _tiled_copy(s, n, lambda r0, nr: pltpu.sync_copy(
                        zeros_hbm.at[pl.ds(0, nr)], acc.at[pl.ds(r0, nr)]))
                    plsc.subcore_barrier()

                    def bat(i, carry, p=p, nblk=nblk):
                        bs = [(i * K + k) * _NS + s for k in range(K)]
                        for k in range(K):
                            @pl.when(bs[k] < nblk)
                            def _(k=k):
                                pltpu.async_copy(
                                    idx_hbm[p].at[pl.ds(bs[k] * _EB, _EB)],
                                    idx_v[k], semi)
                        for k in range(K):
                            @pl.when(bs[k] < nblk)
                            def _(k=k):
                                pltpu.make_async_copy(
                                    idx_hbm[p].at[pl.ds(bs[k] * _EB, _EB)],
                                    idx_v[k], semi).wait()
                                pltpu.async_copy(
                                    ones_v, acc.at[idx_v[k]], sems, add=True)
                        for k in range(K):
                            @pl.when(bs[k] < nblk)
                            def _(k=k):
                                pltpu.make_async_copy(
                                    ones_v, acc.at[idx_v[k]], sems).wait()
                        return carry

                    lax.fori_loop(0, nbat, bat, 0)
                    plsc.subcore_barrier()
                    _tiled_copy(s, n, lambda r0, nr, p=p: pltpu.sync_copy(
                        acc.at[pl.ds(r0, nr)], outs[p].at[pl.ds(r0, nr)]))
                    plsc.subcore_barrier()

    kfn = pl.kernel(
        body,
        out_type=[jax.ShapeDtypeStruct((n, 16), _F32) for n in n_nodes],
        mesh=_sc_mesh(),
        compiler_params=pltpu.CompilerParams(use_tc_tiling_on_sc=False),
        scratch_types=[
            pltpu.VMEM_SHARED((max_n, 16), _F32),
        ] + [pltpu.VMEM((_EB,), jnp.int32) for _ in range(K)] + [
            pltpu.VMEM((_EB, 16), _F32),
            pltpu.SemaphoreType.DMA,
            pltpu.SemaphoreType.DMA,
        ],
    )
    return kfn(*idx_arrays, zeros, ones_blk)


def _sc_gather_scatter(src, dst, tables, n_dst, width):
    k = len(tables)
    half = k // 2
    splits = (tuple(range(half)), tuple(range(half, k)))
    nblk = src.shape[0] // _EB
    per = -(-nblk // _NS)
    zeros = jnp.zeros((_row_part(n_dst)[0], width), _F32)

    K = 3 if width > 128 else 4

    def body(*refs):
        src_hbm, dst_hbm = refs[0], refs[1]
        tabs = refs[2:2 + k]
        zeros_hbm = refs[2 + k]
        outs = refs[3 + k:3 + 2 * k]
        acc = refs[3 + 2 * k]
        idx_s = refs[4 + 2 * k:4 + 2 * k + K]
        idx_d = refs[4 + 2 * k + K:4 + 2 * k + 2 * K]
        gbuf = refs[4 + 2 * k + 2 * K:4 + 2 * k + 3 * K]
        semi, semg, sems = refs[4 + 2 * k + 3 * K:]
        c = lax.axis_index("c")
        s = lax.axis_index("s")
        for ci, chunks in enumerate(splits):
            @pl.when(c == ci)
            def _(chunks=chunks):
                for ch in chunks:
                    _tiled_copy(s, n_dst, lambda r0, nr: pltpu.sync_copy(
                        zeros_hbm.at[pl.ds(0, nr)], acc.at[pl.ds(r0, nr)]))
                    plsc.subcore_barrier()
                    nbat = -(-per // K)

                    def bat(i, carry, ch=ch):
                        bs = [(i * K + kk) * _NS + s for kk in range(K)]
                        for kk in range(K):
                            @pl.when(bs[kk] < nblk)
                            def _(kk=kk):
                                pltpu.async_copy(
                                    src_hbm.at[pl.ds(bs[kk] * _EB, _EB)],
                                    idx_s[kk], semi)
                                pltpu.async_copy(
                                    dst_hbm.at[pl.ds(bs[kk] * _EB, _EB)],
                                    idx_d[kk], semi)
                        for kk in range(K):
                            @pl.when(bs[kk] < nblk)
                            def _(kk=kk):
                                pltpu.make_async_copy(
                                    src_hbm.at[pl.ds(bs[kk] * _EB, _EB)],
                                    idx_s[kk], semi).wait()
                                pltpu.make_async_copy(
                                    dst_hbm.at[pl.ds(bs[kk] * _EB, _EB)],
                                    idx_d[kk], semi).wait()
                                pltpu.async_copy(
                                    tabs[ch].at[idx_s[kk]], gbuf[kk], semg)
                        for kk in range(K):
                            @pl.when(bs[kk] < nblk)
                            def _(kk=kk):
                                pltpu.make_async_copy(
                                    tabs[ch].at[idx_s[kk]], gbuf[kk],
                                    semg).wait()
                                pltpu.async_copy(
                                    gbuf[kk], acc.at[idx_d[kk]], sems,
                                    add=True)
                        for kk in range(K):
                            @pl.when(bs[kk] < nblk)
                            def _(kk=kk):
                                pltpu.make_async_copy(
                                    gbuf[kk], acc.at[idx_d[kk]], sems).wait()
                        return carry

                    lax.fori_loop(0, nbat, bat, 0)
                    plsc.subcore_barrier()
                    _tiled_copy(s, n_dst, lambda r0, nr, ch=ch: pltpu.sync_copy(
                        acc.at[pl.ds(r0, nr)], outs[ch].at[pl.ds(r0, nr)]))
                    plsc.subcore_barrier()

    kfn = pl.kernel(
        body,
        out_type=[jax.ShapeDtypeStruct((n_dst, width), _F32)
                  for _ in range(k)],
        mesh=_sc_mesh(),
        compiler_params=pltpu.CompilerParams(use_tc_tiling_on_sc=False),
        scratch_types=[
            pltpu.VMEM_SHARED((n_dst, width), _F32),
        ] + [pltpu.VMEM((_EB,), jnp.int32) for _ in range(2 * K)] + [
            pltpu.VMEM((_EB, width), _F32) for _ in range(K)] + [
            pltpu.SemaphoreType.DMA,
            pltpu.SemaphoreType.DMA,
            pltpu.SemaphoreType.DMA,
        ],
    )
    return kfn(src, dst, *tables, zeros)


def _rs(col):
    return lax.rsqrt(jnp.maximum(col, 1.0))


def _tc_prescale0(x0, deg_src0):
    n = x0.shape[0]
    br = 1000

    def body(x_ref, dg_ref, o0, o1, o2, o3):
        sc = _rs(dg_ref[:, 0])[:, None]
        outs = (o0, o1, o2, o3)
        for ci in range(4):
            outs[ci][...] = x_ref[:, ci * 32:(ci + 1) * 32] * sc

    return pl.pallas_call(
        body,
        grid=(n // br,),
        in_specs=[pl.BlockSpec((br, 128), lambda i: (i, 0)),
                  pl.BlockSpec((br, 16), lambda i: (i, 0))],
        out_specs=[pl.BlockSpec((br, 32), lambda i: (i, 0))] * 4,
        out_shape=[jax.ShapeDtypeStruct((n, 32), _F32)] * 4,
    )(x0, deg_src0)


def _tc_layer1(aggs, deg_dst0, deg_src1, Wcat, bcat):
    n = aggs[0].shape[0]
    br = 1000

    def body(a0, a1, a2, a3, dd, dsr, W, bb, y0, y1, y2, y3, s12, s13):
        i = pl.program_id(0)
        X = jnp.concatenate([a0[...], a1[...], a2[...], a3[...]], axis=1)
        sd = _rs(dd[:, 0])[:, None]
        H = jnp.dot(X * sd, W[...], preferred_element_type=_F32) + bb[...]
        H = jnp.maximum(H, 0.0)
        x11 = H[:, :128]
        x12 = H[:, 128:256]
        x13 = H[:, 256:]
        sum1 = x11 + x12
        sc = _rs(dsr[:, 0])[:, None]
        y0[...] = sum1[:, :64] * sc
        y1[...] = sum1[:, 64:] * sc
        y2[...] = x12[:, :64] * sc
        y3[...] = x12[:, 64:] * sc

        @pl.when(i == 0)
        def _():
            s12[...] = jnp.zeros((1, 1), _F32)
            s13[...] = jnp.zeros((1, 1), _F32)
        s12[...] = s12[...] + jnp.sum(x12)
        s13[...] = s13[...] + jnp.sum(x13)

    scal = lambda i: (0, 0)
    return pl.pallas_call(
        body,
        grid=(n // br,),
        in_specs=[pl.BlockSpec((br, 32), lambda i: (i, 0))] * 4 + [
            pl.BlockSpec((br, 16), lambda i: (i, 0)),
            pl.BlockSpec((br, 16), lambda i: (i, 0)),
            pl.BlockSpec((128, 384), scal),
            pl.BlockSpec((1, 384), scal),
        ],
        out_specs=[pl.BlockSpec((br, 64), lambda i: (i, 0))] * 4 + [
            pl.BlockSpec((1, 1), scal), pl.BlockSpec((1, 1), scal)],
        out_shape=[jax.ShapeDtypeStruct((n, 64), _F32)] * 4 + [
            jax.ShapeDtypeStruct((1, 1), _F32)] * 2,
    )(*aggs, deg_dst0, deg_src1, Wcat, bcat)


def _tc_layer2(aggs, deg_dst1, deg_src2, W11, b11, W12, b12):
    n = aggs[0].shape[0]
    br = 1000

    def body(a0, a1, a2, a3, dd, dsr, Wa, ba, Wb, bb, z0, z1):
        sd = _rs(dd[:, 0])[:, None]
        AS = jnp.concatenate([a0[...], a1[...]], axis=1) * sd
        AX = jnp.concatenate([a2[...], a3[...]], axis=1) * sd
        x21 = jnp.maximum(
            jnp.dot(AS, Wa[...], preferred_element_type=_F32) + ba[...], 0.0)
        x22 = jnp.maximum(
            jnp.dot(AX, Wb[...], preferred_element_type=_F32) + bb[...], 0.0)
        sc = _rs(dsr[:, 0])[:, None]
        aux = jnp.concatenate([sc, jnp.zeros((br, 31), _F32)], axis=1)
        full = jnp.concatenate([x21 * sc, x22 * sc, aux], axis=1)
        z0[...] = full[:, :144]
        z1[...] = full[:, 144:]

    scal = lambda i: (0, 0)
    return pl.pallas_call(
        body,
        grid=(n // br,),
        in_specs=[pl.BlockSpec((br, 64), lambda i: (i, 0))] * 4 + [
            pl.BlockSpec((br, 16), lambda i: (i, 0)),
            pl.BlockSpec((br, 16), lambda i: (i, 0)),
            pl.BlockSpec((128, 128), scal), pl.BlockSpec((1, 128), scal),
            pl.BlockSpec((128, 128), scal), pl.BlockSpec((1, 128), scal),
        ],
        out_specs=[pl.BlockSpec((br, 144), lambda i: (i, 0))] * 2,
        out_shape=[jax.ShapeDtypeStruct((n, 144), _F32)] * 2,
    )(*aggs, deg_dst1, deg_src2, W11, b11, W12, b12)


def _tc_layer3(o0, o1, deg_dst2, deg_src3, W21, b21, W22, b22, s12):
    n = o0.shape[0]
    br = 800

    def body(r0, r1, dd, dsr, Wa, ba, Wb, bb, sm, w0, w1):
        cat = jnp.concatenate([r0[...], r1[...]], axis=1)
        sd = _rs(dd[:, 0])[:, None]
        aggA = cat[:, :128] * sd
        aggB = cat[:, 128:256] * sd
        n2 = cat[:, 256:257] * sd
        c1 = sm[...][0, 0] * (1.0 / (float(_N1) * 128.0))
        x31 = jnp.maximum(
            jnp.dot(aggA + aggB, Wa[...], preferred_element_type=_F32)
            + ba[...], 0.0)
        x32 = jnp.maximum(
            jnp.dot(aggA + c1 * n2, Wb[...], preferred_element_type=_F32)
            + bb[...], 0.0)
        S = x31 + x32
        sc3 = _rs(dsr[:, 0])[:, None]
        aux = jnp.concatenate([sc3, jnp.zeros((br, 31), _F32)], axis=1)
        full = jnp.concatenate([S * sc3, aux], axis=1)
        w0[...] = full[:, :80]
        w1[...] = full[:, 80:]

    scal = lambda i: (0, 0)
    return pl.pallas_call(
        body,
        grid=(n // br,),
        in_specs=[pl.BlockSpec((br, 144), lambda i: (i, 0))] * 2 + [
            pl.BlockSpec((br, 16), lambda i: (i, 0)),
            pl.BlockSpec((br, 16), lambda i: (i, 0)),
            pl.BlockSpec((128, 128), scal), pl.BlockSpec((1, 128), scal),
            pl.BlockSpec((128, 128), scal), pl.BlockSpec((1, 128), scal),
            pl.BlockSpec((1, 1), scal),
        ],
        out_specs=[pl.BlockSpec((br, 80), lambda i: (i, 0))] * 2,
        out_shape=[jax.ShapeDtypeStruct((n, 80), _F32)] * 2,
    )(o0, o1, deg_dst2, deg_src3, W21, b21, W22, b22, s12)


def _tc_final(d0, d1, deg_dst3, W3, b3, s13):
    n = d0.shape[0]

    def body(r0, r1, dd, W, bb, sm, o):
        cat = jnp.concatenate([r0[...], r1[...]], axis=1)
        sd = _rs(dd[:, 0])[:, None]
        F = cat[:, :128] * sd
        n3 = cat[:, 128:129] * sd
        c2 = sm[...][0, 0] * (1.0 / (float(_N1) * 128.0))
        o[...] = jnp.maximum(
            jnp.dot(F + c2 * n3, W[...], preferred_element_type=_F32)
            + bb[...], 0.0)

    scal = lambda: (0, 0)
    return pl.pallas_call(
        body,
        in_specs=[pl.BlockSpec((n, 80), scal)] * 2 + [
            pl.BlockSpec((n, 16), scal),
            pl.BlockSpec((128, 128), scal), pl.BlockSpec((1, 128), scal),
            pl.BlockSpec((1, 1), scal),
        ],
        out_specs=pl.BlockSpec((n, 128), scal),
        out_shape=jax.ShapeDtypeStruct((n, 128), _F32),
    )(d0, d1, deg_dst3, W3, b3, s13)


def kernel(x0, src0, dst0, src1, dst1, src2, dst2, src3, dst3,
           W01, b01, W02, b02, W03, b03, W11, b11, W12, b12,
           W21, b21, W22, b22, W3, b3):
    degs = _sc_degrees(
        [src0, src1, src2, src3, dst0, dst1, dst2, dst3],
        [_N0, _N1, _N2, _N3, _N1, _N2, _N3, _N4])
    dsrc0, dsrc1, dsrc2, dsrc3, ddst0, ddst1, ddst2, ddst3 = degs

    xt = _tc_prescale0(x0, dsrc0)
    agg0 = _sc_gather_scatter(src0, dst0, xt, _N1, 32)

    Wcat = jnp.concatenate([W01, W02, W03], axis=1)
    bcat = jnp.concatenate([b01, b02, b03]).reshape(1, 384)
    y0, y1, y2, y3, s12, s13 = _tc_layer1(agg0, ddst0, dsrc1, Wcat, bcat)

    agg1 = _sc_gather_scatter(src1, dst1, (y0, y1, y2, y3), _N2, 64)
    zt = _tc_layer2(agg1, ddst1, dsrc2,
                    W11, b11.reshape(1, 128), W12, b12.reshape(1, 128))

    agg2 = _sc_gather_scatter(src2, dst2, zt, _N3, 144)
    wt = _tc_layer3(agg2[0], agg2[1], ddst2, dsrc3,
                    W21, b21.reshape(1, 128), W22, b22.reshape(1, 128), s12)

    agg3 = _sc_gather_scatter(src3, dst3, wt, _N4, 80)
    return _tc_final(agg3[0], agg3[1], ddst3, W3, b3.reshape(1, 128), s13)


# speedup vs baseline: 5.7819x; 1.0416x over previous
import functools

import jax
import jax.numpy as jnp
from jax import lax
from jax.experimental import pallas as pl
from jax.experimental.pallas import tpu as pltpu
from jax.experimental.pallas import tpu_sc as plsc

_F32 = jnp.float32
_NC = 2
_NS = 16
_EB = 128

_N0, _N1, _N2, _N3, _N4 = 100000, 40000, 16000, 6400, 2560


def _sc_mesh():
    return plsc.VectorSubcoreMesh(
        core_axis_name="c", subcore_axis_name="s",
        num_cores=_NC, num_subcores=_NS)


def _row_part(n):
    rpt = -(-(-(-n // _NS)) // 8) * 8
    last = n - (_NS - 1) * rpt
    assert last > 0
    return rpt, last


def _tiled_copy(s, n, copy_fn):
    rpt, last = _row_part(n)
    if last == rpt:
        copy_fn(s * rpt, rpt)
    else:
        @pl.when(s < _NS - 1)
        def _():
            copy_fn(s * rpt, rpt)

        @pl.when(s == _NS - 1)
        def _():
            copy_fn((_NS - 1) * rpt, last)


def _sc_degrees(idx_arrays, n_nodes):
    n_arr = len(idx_arrays)
    half = n_arr // 2
    splits = (tuple(range(half)), tuple(range(half, n_arr)))
    max_n = max(n_nodes)
    max_rpt = max(_row_part(n)[0] for n in n_nodes)
    zeros = jnp.zeros((max_rpt, 16), _F32)
    ones_blk = (lax.broadcasted_iota(jnp.int32, (_EB, 16), 1) == 0).astype(_F32)

    K = 8

    def body(*refs):
        idx_hbm = refs[0:n_arr]
        zeros_hbm = refs[n_arr]
        ones_hbm = refs[n_arr + 1]
        outs = refs[n_arr + 2:n_arr + 2 + n_arr]
        acc = refs[2 * n_arr + 2]
        idx_v = refs[2 * n_arr + 3:2 * n_arr + 3 + K]
        ones_v, semi, sems = refs[2 * n_arr + 3 + K:]
        c = lax.axis_index("c")
        s = lax.axis_index("s")
        pltpu.sync_copy(ones_hbm, ones_v)
        for ci, passes in enumerate(splits):
            @pl.when(c == ci)
            def _(passes=passes):
                for p in passes:
                    n = n_nodes[p]
                    nblk = idx_arrays[p].shape[0] // _EB
                    per = -(-nblk // _NS)
                    nbat = -(-per // K)
                    _tiled_copy(s, n, lambda r0, nr: pltpu.sync_copy(
                        zeros_hbm.at[pl.ds(0, nr)], acc.at[pl.ds(r0, nr)]))
                    plsc.subcore_barrier()

                    def bat(i, carry, p=p, nblk=nblk):
                        bs = [(i * K + k) * _NS + s for k in range(K)]
                        for k in range(K):
                            @pl.when(bs[k] < nblk)
                            def _(k=k):
                                pltpu.async_copy(
                                    idx_hbm[p].at[pl.ds(bs[k] * _EB, _EB)],
                                    idx_v[k], semi)
                        for k in range(K):
                            @pl.when(bs[k] < nblk)
                            def _(k=k):
                                pltpu.make_async_copy(
                                    idx_hbm[p].at[pl.ds(bs[k] * _EB, _EB)],
                                    idx_v[k], semi).wait()
                                pltpu.async_copy(
                                    ones_v, acc.at[idx_v[k]], sems, add=True)
                        for k in range(K):
                            @pl.when(bs[k] < nblk)
                            def _(k=k):
                                pltpu.make_async_copy(
                                    ones_v, acc.at[idx_v[k]], sems).wait()
                        return carry

                    lax.fori_loop(0, nbat, bat, 0)
                    plsc.subcore_barrier()
                    _tiled_copy(s, n, lambda r0, nr, p=p: pltpu.sync_copy(
                        acc.at[pl.ds(r0, nr)], outs[p].at[pl.ds(r0, nr)]))
                    plsc.subcore_barrier()

    kfn = pl.kernel(
        body,
        out_type=[jax.ShapeDtypeStruct((n, 16), _F32) for n in n_nodes],
        mesh=_sc_mesh(),
        compiler_params=pltpu.CompilerParams(use_tc_tiling_on_sc=False),
        scratch_types=[
            pltpu.VMEM_SHARED((max_n, 16), _F32),
        ] + [pltpu.VMEM((_EB,), jnp.int32) for _ in range(K)] + [
            pltpu.VMEM((_EB, 16), _F32),
            pltpu.SemaphoreType.DMA,
            pltpu.SemaphoreType.DMA,
        ],
    )
    return kfn(*idx_arrays, zeros, ones_blk)


def _sc_gather_scatter(src, dst, tables, n_dst, width):
    k = len(tables)
    half = k // 2
    splits = (tuple(range(half)), tuple(range(half, k)))
    nblk = src.shape[0] // _EB
    per = -(-nblk // _NS)
    zeros = jnp.zeros((_row_part(n_dst)[0], width), _F32)

    K = max(1, min(8, (2097151 - n_dst * width - 65536)
                   // (_NS * _EB * width)))

    def body(*refs):
        src_hbm, dst_hbm = refs[0], refs[1]
        tabs = refs[2:2 + k]
        zeros_hbm = refs[2 + k]
        outs = refs[3 + k:3 + 2 * k]
        acc = refs[3 + 2 * k]
        idx_s = refs[4 + 2 * k:4 + 2 * k + K]
        idx_d = refs[4 + 2 * k + K:4 + 2 * k + 2 * K]
        gbuf = refs[4 + 2 * k + 2 * K:4 + 2 * k + 3 * K]
        semi, semg, sems = refs[4 + 2 * k + 3 * K:]
        c = lax.axis_index("c")
        s = lax.axis_index("s")
        for ci, chunks in enumerate(splits):
            @pl.when(c == ci)
            def _(chunks=chunks):
                for ch in chunks:
                    _tiled_copy(s, n_dst, lambda r0, nr: pltpu.sync_copy(
                        zeros_hbm.at[pl.ds(0, nr)], acc.at[pl.ds(r0, nr)]))
                    plsc.subcore_barrier()
                    nbat = -(-per // K)

                    def bat(i, carry, ch=ch):
                        bs = [(i * K + kk) * _NS + s for kk in range(K)]
                        for kk in range(K):
                            @pl.when(bs[kk] < nblk)
                            def _(kk=kk):
                                pltpu.async_copy(
                                    src_hbm.at[pl.ds(bs[kk] * _EB, _EB)],
                                    idx_s[kk], semi)
                                pltpu.async_copy(
                                    dst_hbm.at[pl.ds(bs[kk] * _EB, _EB)],
                                    idx_d[kk], semi)
                        for kk in range(K):
                            @pl.when(bs[kk] < nblk)
                            def _(kk=kk):
                                pltpu.make_async_copy(
                                    src_hbm.at[pl.ds(bs[kk] * _EB, _EB)],
                                    idx_s[kk], semi).wait()
                                pltpu.make_async_copy(
                                    dst_hbm.at[pl.ds(bs[kk] * _EB, _EB)],
                                    idx_d[kk], semi).wait()
                                pltpu.async_copy(
                                    tabs[ch].at[idx_s[kk]], gbuf[kk], semg)
                        for kk in range(K):
                            @pl.when(bs[kk] < nblk)
                            def _(kk=kk):
                                pltpu.make_async_copy(
                                    tabs[ch].at[idx_s[kk]], gbuf[kk],
                                    semg).wait()
                                pltpu.async_copy(
                                    gbuf[kk], acc.at[idx_d[kk]], sems,
                                    add=True)
                        for kk in range(K):
                            @pl.when(bs[kk] < nblk)
                            def _(kk=kk):
                                pltpu.make_async_copy(
                                    gbuf[kk], acc.at[idx_d[kk]], sems).wait()
                        return carry

                    lax.fori_loop(0, nbat, bat, 0)
                    plsc.subcore_barrier()
                    _tiled_copy(s, n_dst, lambda r0, nr, ch=ch: pltpu.sync_copy(
                        acc.at[pl.ds(r0, nr)], outs[ch].at[pl.ds(r0, nr)]))
                    plsc.subcore_barrier()

    kfn = pl.kernel(
        body,
        out_type=[jax.ShapeDtypeStruct((n_dst, width), _F32)
                  for _ in range(k)],
        mesh=_sc_mesh(),
        compiler_params=pltpu.CompilerParams(use_tc_tiling_on_sc=False),
        scratch_types=[
            pltpu.VMEM_SHARED((n_dst, width), _F32),
        ] + [pltpu.VMEM((_EB,), jnp.int32) for _ in range(2 * K)] + [
            pltpu.VMEM((_EB, width), _F32) for _ in range(K)] + [
            pltpu.SemaphoreType.DMA,
            pltpu.SemaphoreType.DMA,
            pltpu.SemaphoreType.DMA,
        ],
    )
    return kfn(src, dst, *tables, zeros)


def _rs(col):
    return lax.rsqrt(jnp.maximum(col, 1.0))


def _tc_layer1(aggs, deg_dst0, deg_src1, Wcat, bcat):
    n = aggs[0].shape[0]
    br = 1000

    def body(a0, a1, a2, a3, dd, dsr, W, bb, y0, y1, y2, y3, s12, s13):
        i = pl.program_id(0)
        X = jnp.concatenate([a0[...], a1[...], a2[...], a3[...]], axis=1)
        sd = _rs(dd[:, 0])[:, None]
        H = jnp.dot(X * sd, W[...], preferred_element_type=_F32) + bb[...]
        H = jnp.maximum(H, 0.0)
        x11 = H[:, :128]
        x12 = H[:, 128:256]
        x13 = H[:, 256:]
        sum1 = x11 + x12
        sc = _rs(dsr[:, 0])[:, None]
        y0[...] = sum1[:, :64] * sc
        y1[...] = sum1[:, 64:] * sc
        y2[...] = x12[:, :64] * sc
        y3[...] = x12[:, 64:] * sc

        @pl.when(i == 0)
        def _():
            s12[...] = jnp.zeros((1, 1), _F32)
            s13[...] = jnp.zeros((1, 1), _F32)
        s12[...] = s12[...] + jnp.sum(x12)
        s13[...] = s13[...] + jnp.sum(x13)

    scal = lambda i: (0, 0)
    return pl.pallas_call(
        body,
        grid=(n // br,),
        in_specs=[pl.BlockSpec((br, 32), lambda i: (i, 0))] * 4 + [
            pl.BlockSpec((br, 16), lambda i: (i, 0)),
            pl.BlockSpec((br, 16), lambda i: (i, 0)),
            pl.BlockSpec((128, 384), scal),
            pl.BlockSpec((1, 384), scal),
        ],
        out_specs=[pl.BlockSpec((br, 64), lambda i: (i, 0))] * 4 + [
            pl.BlockSpec((1, 1), scal), pl.BlockSpec((1, 1), scal)],
        out_shape=[jax.ShapeDtypeStruct((n, 64), _F32)] * 4 + [
            jax.ShapeDtypeStruct((1, 1), _F32)] * 2,
    )(*aggs, deg_dst0, deg_src1, Wcat, bcat)


def _tc_layer2(aggs, deg_dst1, deg_src2, W11, b11, W12, b12):
    n = aggs[0].shape[0]
    br = 1000

    def body(a0, a1, a2, a3, dd, dsr, Wa, ba, Wb, bb, z0, z1):
        sd = _rs(dd[:, 0])[:, None]
        AS = jnp.concatenate([a0[...], a1[...]], axis=1) * sd
        AX = jnp.concatenate([a2[...], a3[...]], axis=1) * sd
        x21 = jnp.maximum(
            jnp.dot(AS, Wa[...], preferred_element_type=_F32) + ba[...], 0.0)
        x22 = jnp.maximum(
            jnp.dot(AX, Wb[...], preferred_element_type=_F32) + bb[...], 0.0)
        sc = _rs(dsr[:, 0])[:, None]
        aux = jnp.concatenate([sc, jnp.zeros((br, 31), _F32)], axis=1)
        full = jnp.concatenate([x21 * sc, x22 * sc, aux], axis=1)
        z0[...] = full[:, :144]
        z1[...] = full[:, 144:]

    scal = lambda i: (0, 0)
    return pl.pallas_call(
        body,
        grid=(n // br,),
        in_specs=[pl.BlockSpec((br, 64), lambda i: (i, 0))] * 4 + [
            pl.BlockSpec((br, 16), lambda i: (i, 0)),
            pl.BlockSpec((br, 16), lambda i: (i, 0)),
            pl.BlockSpec((128, 128), scal), pl.BlockSpec((1, 128), scal),
            pl.BlockSpec((128, 128), scal), pl.BlockSpec((1, 128), scal),
        ],
        out_specs=[pl.BlockSpec((br, 144), lambda i: (i, 0))] * 2,
        out_shape=[jax.ShapeDtypeStruct((n, 144), _F32)] * 2,
    )(*aggs, deg_dst1, deg_src2, W11, b11, W12, b12)


def _tc_layer3(o0, o1, deg_dst2, deg_src3, W21, b21, W22, b22, s12):
    n = o0.shape[0]
    br = 800

    def body(r0, r1, dd, dsr, Wa, ba, Wb, bb, sm, w0, w1):
        cat = jnp.concatenate([r0[...], r1[...]], axis=1)
        sd = _rs(dd[:, 0])[:, None]
        aggA = cat[:, :128] * sd
        aggB = cat[:, 128:256] * sd
        n2 = cat[:, 256:257] * sd
        c1 = sm[...][0, 0] * (1.0 / (float(_N1) * 128.0))
        x31 = jnp.maximum(
            jnp.dot(aggA + aggB, Wa[...], preferred_element_type=_F32)
            + ba[...], 0.0)
        x32 = jnp.maximum(
            jnp.dot(aggA + c1 * n2, Wb[...], preferred_element_type=_F32)
            + bb[...], 0.0)
        S = x31 + x32
        sc3 = _rs(dsr[:, 0])[:, None]
        aux = jnp.concatenate([sc3, jnp.zeros((br, 31), _F32)], axis=1)
        full = jnp.concatenate([S * sc3, aux], axis=1)
        w0[...] = full[:, :80]
        w1[...] = full[:, 80:]

    scal = lambda i: (0, 0)
    return pl.pallas_call(
        body,
        grid=(n // br,),
        in_specs=[pl.BlockSpec((br, 144), lambda i: (i, 0))] * 2 + [
            pl.BlockSpec((br, 16), lambda i: (i, 0)),
            pl.BlockSpec((br, 16), lambda i: (i, 0)),
            pl.BlockSpec((128, 128), scal), pl.BlockSpec((1, 128), scal),
            pl.BlockSpec((128, 128), scal), pl.BlockSpec((1, 128), scal),
            pl.BlockSpec((1, 1), scal),
        ],
        out_specs=[pl.BlockSpec((br, 80), lambda i: (i, 0))] * 2,
        out_shape=[jax.ShapeDtypeStruct((n, 80), _F32)] * 2,
    )(o0, o1, deg_dst2, deg_src3, W21, b21, W22, b22, s12)


def _tc_final(d0, d1, deg_dst3, W3, b3, s13):
    n = d0.shape[0]

    def body(r0, r1, dd, W, bb, sm, o):
        cat = jnp.concatenate([r0[...], r1[...]], axis=1)
        sd = _rs(dd[:, 0])[:, None]
        F = cat[:, :128] * sd
        n3 = cat[:, 128:129] * sd
        c2 = sm[...][0, 0] * (1.0 / (float(_N1) * 128.0))
        o[...] = jnp.maximum(
            jnp.dot(F + c2 * n3, W[...], preferred_element_type=_F32)
            + bb[...], 0.0)

    scal = lambda: (0, 0)
    return pl.pallas_call(
        body,
        in_specs=[pl.BlockSpec((n, 80), scal)] * 2 + [
            pl.BlockSpec((n, 16), scal),
            pl.BlockSpec((128, 128), scal), pl.BlockSpec((1, 128), scal),
            pl.BlockSpec((1, 1), scal),
        ],
        out_specs=pl.BlockSpec((n, 128), scal),
        out_shape=jax.ShapeDtypeStruct((n, 128), _F32),
    )(d0, d1, deg_dst3, W3, b3, s13)


def kernel(x0, src0, dst0, src1, dst1, src2, dst2, src3, dst3,
           W01, b01, W02, b02, W03, b03, W11, b11, W12, b12,
           W21, b21, W22, b22, W3, b3):
    degs = _sc_degrees(
        [src0, src1, src2, src3, dst0, dst1, dst2, dst3],
        [_N0, _N1, _N2, _N3, _N1, _N2, _N3, _N4])
    dsrc0, dsrc1, dsrc2, dsrc3, ddst0, ddst1, ddst2, ddst3 = degs

    s0 = lax.rsqrt(jnp.maximum(dsrc0[:, 0], 1.0))[:, None]
    xt = [x0[:, c * 32:(c + 1) * 32] * s0 for c in range(4)]
    agg0 = _sc_gather_scatter(src0, dst0, xt, _N1, 32)

    Wcat = jnp.concatenate([W01, W02, W03], axis=1)
    bcat = jnp.concatenate([b01, b02, b03]).reshape(1, 384)
    y0, y1, y2, y3, s12, s13 = _tc_layer1(agg0, ddst0, dsrc1, Wcat, bcat)

    agg1 = _sc_gather_scatter(src1, dst1, (y0, y1, y2, y3), _N2, 64)
    zt = _tc_layer2(agg1, ddst1, dsrc2,
                    W11, b11.reshape(1, 128), W12, b12.reshape(1, 128))

    agg2 = _sc_gather_scatter(src2, dst2, zt, _N3, 144)
    wt = _tc_layer3(agg2[0], agg2[1], ddst2, dsrc3,
                    W21, b21.reshape(1, 128), W22, b22.reshape(1, 128), s12)

    agg3 = _sc_gather_scatter(src3, dst3, wt, _N4, 80)
    return _tc_final(agg3[0], agg3[1], ddst3, W3, b3.reshape(1, 128), s13)
